# Initial kernel scaffold; baseline (speedup 1.0000x reference)
import jax
import jax.numpy as jnp
from jax.experimental import pallas as pl


def kernel(tgt, src, reference_points, spatial_shapes, level_start_index, W_off, b_off, W_attn, b_attn, W_val, b_val, W_out, b_out, g1, b1n, W1, bl1, W2, bl2, g2, b2n):
    raise NotImplementedError("write your pallas kernel here")


# speedup vs baseline: 3.7179x; 3.7179x over previous
import functools

import jax
import jax.numpy as jnp
import numpy as np
from jax import lax
from jax.experimental import pallas as pl
from jax.experimental.pallas import tpu as pltpu
from jax.experimental.pallas import tpu_sc as plsc

B, LQ, D = 4, 1024, 256
H, P = 8, 4
HH, WW = 100, 100
LIN = HH * WW
DH = D // H
NPTS = P * 4
NOUT = B * LQ * H

_E = np.zeros((64, 32), np.float32)
_E[np.arange(32) * 2, np.arange(32)] = 1.0
_O = np.zeros((64, 32), np.float32)
_O[np.arange(32) * 2 + 1, np.arange(32)] = 1.0
_S = np.zeros((32, 32), np.float32)
for _i in range(32):
    for _j in range(32):
        if _i // 4 == _j // 4:
            _S[_i, _j] = 1.0
_PBIG = np.zeros((128, 128), np.float32)
for _c in range(4):
    for _h in range(H):
        for _p in range(P):
            _PBIG[_c * 32 + _h * 4 + _p, _h * 16 + _p * 4 + _c] = 1.0

_VBLK = 2000


def _valproj_body(src_ref, w_ref, b_ref, out_ref):
    out_ref[:] = (
        jnp.dot(src_ref[:], w_ref[:], preferred_element_type=jnp.float32,
                precision=lax.Precision.HIGHEST)
        + b_ref[:]
    )


def _valproj(src_f, W_val, b_val):
    grid = (B * LIN) // _VBLK
    return pl.pallas_call(
        _valproj_body,
        grid=(grid,),
        in_specs=[
            pl.BlockSpec((_VBLK, D), lambda i: (i, 0)),
            pl.BlockSpec((D, D), lambda i: (0, 0)),
            pl.BlockSpec((1, D), lambda i: (0, 0)),
        ],
        out_specs=pl.BlockSpec((_VBLK, D), lambda i: (i, 0)),
        out_shape=jax.ShapeDtypeStruct((B * LIN, D), jnp.float32),
    )(src_f, W_val, b_val)


_QBLK = 256


def _prep_body(tgt_ref, rp_ref, woff_ref, boff_ref, wattn_ref, battn_ref,
               e_ref, o_ref, s_ref, pbig_ref, w_out_ref, idx_out_ref):
    i = pl.program_id(0)
    b_f = (i // (LQ // _QBLK)).astype(jnp.float32)
    t = tgt_ref[:]
    hp = lax.Precision.HIGHEST
    off = jnp.dot(t, woff_ref[:], preferred_element_type=jnp.float32, precision=hp) + boff_ref[:]
    a = jnp.dot(t, wattn_ref[:], preferred_element_type=jnp.float32, precision=hp) + battn_ref[:]
    a = a - jnp.max(a, axis=1, keepdims=True)
    ea = jnp.exp(a)
    gs = jnp.dot(ea, s_ref[:], preferred_element_type=jnp.float32, precision=hp)
    aw = ea / gs

    lane64 = lax.broadcasted_iota(jnp.int32, (_QBLK, 64), 1)
    is_x = (lane64 % 2) == 0
    rp = rp_ref[:]
    refc = jnp.where(is_x, rp[:, 0:1], rp[:, 1:2])
    loc = (refc + off / 100.0) * 100.0 - 0.5
    fl = jnp.floor(loc)
    w1 = loc - fl
    w0 = 1.0 - w1
    v0 = ((fl >= 0.0) & (fl <= 99.0)).astype(jnp.float32)
    v1 = ((fl >= -1.0) & (fl <= 98.0)).astype(jnp.float32)
    c0 = jnp.clip(fl, 0.0, 99.0)
    c1 = jnp.clip(fl + 1.0, 0.0, 99.0)

    E = e_ref[:]
    O = o_ref[:]

    def sel(m, M):
        return jnp.dot(m, M, preferred_element_type=jnp.float32,
                       precision=lax.Precision.HIGHEST)

    wx0, wx1 = sel(w0, E), sel(w1, E)
    wy0, wy1 = sel(w0, O), sel(w1, O)
    vx0, vx1 = sel(v0, E), sel(v1, E)
    vy0, vy1 = sel(v0, O), sel(v1, O)
    cx0, cx1 = sel(c0, E), sel(c1, E)
    cy0, cy1 = sel(c0, O), sel(c1, O)

    wcat = jnp.concatenate(
        [aw * wy0 * wx0 * vy0 * vx0, aw * wy0 * wx1 * vy0 * vx1,
         aw * wy1 * wx0 * vy1 * vx0, aw * wy1 * wx1 * vy1 * vx1], axis=1)
    ycat = jnp.concatenate([cy0, cy0, cy1, cy1], axis=1)
    xcat = jnp.concatenate([cx0, cx1, cx0, cx1], axis=1)
    PB = pbig_ref[:]
    y128 = jnp.dot(ycat, PB, preferred_element_type=jnp.float32, precision=hp)
    x128 = jnp.dot(xcat, PB, preferred_element_type=jnp.float32, precision=hp)
    h128 = (lax.broadcasted_iota(jnp.int32, (_QBLK, 128), 1) // 16).astype(jnp.float32)
    base = b_f * float(LIN * H)
    idxf = (y128 * 100.0 + x128) * 8.0 + h128 + base
    w_out_ref[:] = jnp.dot(wcat, PB, preferred_element_type=jnp.float32, precision=hp)
    idx_out_ref[:] = idxf.astype(jnp.int32)


def _prep(tgt_f, rp_f, W_off, b_off, W_attn, b_attn):
    grid = (B * LQ) // _QBLK
    return pl.pallas_call(
        _prep_body,
        grid=(grid,),
        in_specs=[
            pl.BlockSpec((_QBLK, D), lambda i: (i, 0)),
            pl.BlockSpec((_QBLK, 2), lambda i: (i, 0)),
            pl.BlockSpec((D, 64), lambda i: (0, 0)),
            pl.BlockSpec((1, 64), lambda i: (0, 0)),
            pl.BlockSpec((D, 32), lambda i: (0, 0)),
            pl.BlockSpec((1, 32), lambda i: (0, 0)),
            pl.BlockSpec((64, 32), lambda i: (0, 0)),
            pl.BlockSpec((64, 32), lambda i: (0, 0)),
            pl.BlockSpec((32, 32), lambda i: (0, 0)),
            pl.BlockSpec((128, 128), lambda i: (0, 0)),
        ],
        out_specs=[
            pl.BlockSpec((_QBLK, 128), lambda i: (i, 0)),
            pl.BlockSpec((_QBLK, 128), lambda i: (i, 0)),
        ],
        out_shape=[
            jax.ShapeDtypeStruct((B * LQ, 128), jnp.float32),
            jax.ShapeDtypeStruct((B * LQ, 128), jnp.int32),
        ],
    )(tgt_f, rp_f, W_off, b_off, W_attn, b_attn, _E, _O, _S, _PBIG)


_NW = 32
_RPW = NOUT // _NW
_CH = 64
_NCHUNK = _RPW // _CH
_NGATH = (_CH * NPTS) // 128

def _sc_gather_impl(table_hbm, idx_hbm, w_hbm, out_hbm, idx_v, w_v, rows_v, out_v, sem):
    wid = lax.axis_index("s") * 2 + lax.axis_index("c")
    lanes = jnp.arange(16, dtype=jnp.int32)

    def chunk_body(c, _):
        base = wid * _RPW + c * _CH
        pltpu.sync_copy(idx_hbm.at[pl.ds(wid * 128 + c * _NGATH, _NGATH)], idx_v)
        pltpu.sync_copy(w_hbm.at[pl.ds(base * NPTS, _CH * NPTS)], w_v)
        copies = []
        for j in range(_NGATH):
            copies.append(pltpu.async_copy(
                table_hbm.at[idx_v.at[j]],
                rows_v.at[pl.ds(j * 128, 128)],
                sem,
            ))
        for cp in copies:
            cp.wait()

        def j_body(j, _):
            wv = w_v[pl.ds(j * NPTS, 16)]
            acc0 = jnp.zeros((16,), jnp.float32)
            acc1 = jnp.zeros((16,), jnp.float32)
            for i in range(NPTS):
                k = j * NPTS + i
                g0 = rows_v[k, pl.ds(0, 16)]
                g1 = rows_v[k, pl.ds(16, 16)]
                s = wv[i]
                acc0 = acc0 + g0 * s
                acc1 = acc1 + g1 * s
            out_v[pl.ds(j * DH, 16)] = acc0
            out_v[pl.ds(j * DH + 16, 16)] = acc1
            return 0

        lax.fori_loop(0, _CH, j_body, 0)
        pltpu.sync_copy(out_v, out_hbm.at[pl.ds(base * DH, _CH * DH)])
        return 0

    lax.fori_loop(0, _NCHUNK, chunk_body, 0)


@functools.cache
def _sc_gather_kernel():
    mesh = plsc.VectorSubcoreMesh(
        core_axis_name="c", subcore_axis_name="s", num_cores=2, num_subcores=16)
    return pl.kernel(
        _sc_gather_impl,
        out_type=jax.ShapeDtypeStruct((NOUT * DH,), jnp.float32),
        mesh=mesh,
        scratch_types=[
            pltpu.VMEM((_NGATH, 128), jnp.int32),
            pltpu.VMEM((_CH * NPTS,), jnp.float32),
            pltpu.VMEM((_CH * NPTS, DH), jnp.float32),
            pltpu.VMEM((_CH * DH,), jnp.float32),
            pltpu.SemaphoreType.DMA,
        ],
        compiler_params=pltpu.CompilerParams(use_tc_tiling_on_sc=False),
    )


_FBLK = 512


def _final_body(attn_ref, tgt_ref, w_ref, b_ref, g_ref, bn_ref, out_ref):
    y = (jnp.dot(attn_ref[:], w_ref[:], preferred_element_type=jnp.float32,
                 precision=lax.Precision.HIGHEST)
         + b_ref[:] + tgt_ref[:])
    m = jnp.mean(y, axis=1, keepdims=True)
    yc = y - m
    v = jnp.mean(yc * yc, axis=1, keepdims=True)
    out_ref[:] = yc / jnp.sqrt(v + 1e-5) * g_ref[:] + bn_ref[:]


def _final(attn_f, tgt_f, W_out, b_out, g1, b1n):
    grid = (B * LQ) // _FBLK
    return pl.pallas_call(
        _final_body,
        grid=(grid,),
        in_specs=[
            pl.BlockSpec((_FBLK, D), lambda i: (i, 0)),
            pl.BlockSpec((_FBLK, D), lambda i: (i, 0)),
            pl.BlockSpec((D, D), lambda i: (0, 0)),
            pl.BlockSpec((1, D), lambda i: (0, 0)),
            pl.BlockSpec((1, D), lambda i: (0, 0)),
            pl.BlockSpec((1, D), lambda i: (0, 0)),
        ],
        out_specs=pl.BlockSpec((_FBLK, D), lambda i: (i, 0)),
        out_shape=jax.ShapeDtypeStruct((B * LQ, D), jnp.float32),
    )(attn_f, tgt_f, W_out, b_out, g1, b1n)


def kernel(tgt, src, reference_points, spatial_shapes, level_start_index,
           W_off, b_off, W_attn, b_attn, W_val, b_val, W_out, b_out,
           g1, b1n, W1, bl1, W2, bl2, g2, b2n):
    tgt_f = tgt.reshape(B * LQ, D)
    src_f = src.reshape(B * LIN, D)
    rp_f = reference_points.reshape(B * LQ, 2)

    value = _valproj(src_f, W_val, b_val.reshape(1, D))
    w128, idx128 = _prep(tgt_f, rp_f, W_off, b_off.reshape(1, 64),
                         W_attn, b_attn.reshape(1, 32))

    table = value.reshape(B * LIN * H, DH)
    attn_flat = _sc_gather_kernel()(table, idx128, w128.reshape(-1))

    out = _final(attn_flat.reshape(B * LQ, D), tgt_f,
                 W_out, b_out.reshape(1, D), g1.reshape(1, D), b1n.reshape(1, D))
    return out.reshape(B, LQ, D)


# speedup vs baseline: 4.3230x; 1.1628x over previous
import functools

import jax
import jax.numpy as jnp
import numpy as np
from jax import lax
from jax.experimental import pallas as pl
from jax.experimental.pallas import tpu as pltpu
from jax.experimental.pallas import tpu_sc as plsc

B, LQ, D = 4, 1024, 256
H, P = 8, 4
HH, WW = 100, 100
LIN = HH * WW
DH = D // H
NPTS = P * 4
NOUT = B * LQ * H

_E = np.zeros((64, 32), np.float32)
_E[np.arange(32) * 2, np.arange(32)] = 1.0
_O = np.zeros((64, 32), np.float32)
_O[np.arange(32) * 2 + 1, np.arange(32)] = 1.0
_S = np.zeros((32, 32), np.float32)
for _i in range(32):
    for _j in range(32):
        if _i // 4 == _j // 4:
            _S[_i, _j] = 1.0
_PBIG = np.zeros((128, 128), np.float32)
for _c in range(4):
    for _h in range(H):
        for _p in range(P):
            _PBIG[_c * 32 + _h * 4 + _p, _h * 16 + _p * 4 + _c] = 1.0

_VBLK = 2000


def _valproj_body(src_ref, w_ref, b_ref, out_ref):
    out_ref[:] = (
        jnp.dot(src_ref[:], w_ref[:], preferred_element_type=jnp.float32,
                precision=lax.Precision.HIGHEST)
        + b_ref[:]
    )


def _valproj(src_f, W_val, b_val):
    grid = (B * LIN) // _VBLK
    return pl.pallas_call(
        _valproj_body,
        grid=(grid,),
        in_specs=[
            pl.BlockSpec((_VBLK, D), lambda i: (i, 0)),
            pl.BlockSpec((D, D), lambda i: (0, 0)),
            pl.BlockSpec((1, D), lambda i: (0, 0)),
        ],
        out_specs=pl.BlockSpec((_VBLK, D), lambda i: (i, 0)),
        out_shape=jax.ShapeDtypeStruct((B * LIN, D), jnp.float32),
    )(src_f, W_val, b_val)


_QBLK = 256


def _prep_body(tgt_ref, rp_ref, woff_ref, boff_ref, wattn_ref, battn_ref,
               e_ref, o_ref, s_ref, pbig_ref, w_out_ref, idx_out_ref):
    i = pl.program_id(0)
    b_f = (i // (LQ // _QBLK)).astype(jnp.float32)
    t = tgt_ref[:]
    hp = lax.Precision.HIGHEST
    off = jnp.dot(t, woff_ref[:], preferred_element_type=jnp.float32, precision=hp) + boff_ref[:]
    a = jnp.dot(t, wattn_ref[:], preferred_element_type=jnp.float32, precision=hp) + battn_ref[:]
    a = a - jnp.max(a, axis=1, keepdims=True)
    ea = jnp.exp(a)
    gs = jnp.dot(ea, s_ref[:], preferred_element_type=jnp.float32, precision=hp)
    aw = ea / gs

    lane64 = lax.broadcasted_iota(jnp.int32, (_QBLK, 64), 1)
    is_x = (lane64 % 2) == 0
    rp = rp_ref[:]
    refc = jnp.where(is_x, rp[:, 0:1], rp[:, 1:2])
    loc = (refc + off / 100.0) * 100.0 - 0.5
    fl = jnp.floor(loc)
    w1 = loc - fl
    w0 = 1.0 - w1
    v0 = ((fl >= 0.0) & (fl <= 99.0)).astype(jnp.float32)
    v1 = ((fl >= -1.0) & (fl <= 98.0)).astype(jnp.float32)
    c0 = jnp.clip(fl, 0.0, 99.0)
    c1 = jnp.clip(fl + 1.0, 0.0, 99.0)

    E = e_ref[:]
    O = o_ref[:]

    def sel(m, M):
        return jnp.dot(m, M, preferred_element_type=jnp.float32,
                       precision=lax.Precision.HIGHEST)

    wx0, wx1 = sel(w0, E), sel(w1, E)
    wy0, wy1 = sel(w0, O), sel(w1, O)
    vx0, vx1 = sel(v0, E), sel(v1, E)
    vy0, vy1 = sel(v0, O), sel(v1, O)
    cx0, cx1 = sel(c0, E), sel(c1, E)
    cy0, cy1 = sel(c0, O), sel(c1, O)

    wcat = jnp.concatenate(
        [aw * wy0 * wx0 * vy0 * vx0, aw * wy0 * wx1 * vy0 * vx1,
         aw * wy1 * wx0 * vy1 * vx0, aw * wy1 * wx1 * vy1 * vx1], axis=1)
    ycat = jnp.concatenate([cy0, cy0, cy1, cy1], axis=1)
    xcat = jnp.concatenate([cx0, cx1, cx0, cx1], axis=1)
    PB = pbig_ref[:]
    y128 = jnp.dot(ycat, PB, preferred_element_type=jnp.float32, precision=hp)
    x128 = jnp.dot(xcat, PB, preferred_element_type=jnp.float32, precision=hp)
    h128 = (lax.broadcasted_iota(jnp.int32, (_QBLK, 128), 1) // 16).astype(jnp.float32)
    base = b_f * float(LIN * H)
    idxf = (y128 * 100.0 + x128) * 8.0 + h128 + base
    w_out_ref[:] = jnp.dot(wcat, PB, preferred_element_type=jnp.float32, precision=hp)
    idx_out_ref[:] = idxf.astype(jnp.int32)


def _prep(tgt_f, rp_f, W_off, b_off, W_attn, b_attn):
    grid = (B * LQ) // _QBLK
    return pl.pallas_call(
        _prep_body,
        grid=(grid,),
        in_specs=[
            pl.BlockSpec((_QBLK, D), lambda i: (i, 0)),
            pl.BlockSpec((_QBLK, 2), lambda i: (i, 0)),
            pl.BlockSpec((D, 64), lambda i: (0, 0)),
            pl.BlockSpec((1, 64), lambda i: (0, 0)),
            pl.BlockSpec((D, 32), lambda i: (0, 0)),
            pl.BlockSpec((1, 32), lambda i: (0, 0)),
            pl.BlockSpec((64, 32), lambda i: (0, 0)),
            pl.BlockSpec((64, 32), lambda i: (0, 0)),
            pl.BlockSpec((32, 32), lambda i: (0, 0)),
            pl.BlockSpec((128, 128), lambda i: (0, 0)),
        ],
        out_specs=[
            pl.BlockSpec((_QBLK, 128), lambda i: (i, 0)),
            pl.BlockSpec((_QBLK, 128), lambda i: (i, 0)),
        ],
        out_shape=[
            jax.ShapeDtypeStruct((B * LQ, 128), jnp.float32),
            jax.ShapeDtypeStruct((B * LQ, 128), jnp.int32),
        ],
    )(tgt_f, rp_f, W_off, b_off, W_attn, b_attn, _E, _O, _S, _PBIG)


_NW = 32
_RPW = NOUT // _NW
_CH = 64
_NCHUNK = _RPW // _CH
_NGATH = (_CH * NPTS) // 128

def _sc_gather_impl(table_hbm, idx_hbm, w_hbm, out_hbm,
                    idx_a, idx_b, w_a, w_b, rows_a, rows_b, out_a, out_b,
                    sem_a, sem_b, osem):
    wid = lax.axis_index("s") * 2 + lax.axis_index("c")
    bufs = ((idx_a, w_a, rows_a, out_a, sem_a),
            (idx_b, w_b, rows_b, out_b, sem_b))

    def stage(c, buf):
        idx_v, w_v, rows_v, _, sem = buf
        base = wid * _RPW + c * _CH
        pltpu.sync_copy(idx_hbm.at[pl.ds(wid * 128 + c * _NGATH, _NGATH)], idx_v)
        pltpu.sync_copy(w_hbm.at[pl.ds(base * NPTS, _CH * NPTS)], w_v)
        return [pltpu.async_copy(table_hbm.at[idx_v.at[j]],
                                 rows_v.at[pl.ds(j * 128, 128)], sem)
                for j in range(_NGATH)]

    def compute(c, buf, copies):
        _, w_v, rows_v, out_v, _ = buf
        base = wid * _RPW + c * _CH
        for cp in copies:
            cp.wait()

        def j_body(j, _):
            wv = w_v[pl.ds(j * NPTS, 16)]
            acc0 = jnp.zeros((16,), jnp.float32)
            acc1 = jnp.zeros((16,), jnp.float32)
            for i in range(NPTS):
                k = j * NPTS + i
                g0 = rows_v[k, pl.ds(0, 16)]
                g1 = rows_v[k, pl.ds(16, 16)]
                s = wv[i]
                acc0 = acc0 + g0 * s
                acc1 = acc1 + g1 * s
            out_v[pl.ds(j * DH, 16)] = acc0
            out_v[pl.ds(j * DH + 16, 16)] = acc1
            return 0

        lax.fori_loop(0, _CH, j_body, 0)
        return pltpu.async_copy(out_v, out_hbm.at[pl.ds(base * DH, _CH * DH)], osem)

    out_copies = [None, None]
    copies = stage(0, bufs[0])
    for c in range(_NCHUNK):
        nxt = stage(c + 1, bufs[(c + 1) % 2]) if c + 1 < _NCHUNK else None
        if out_copies[c % 2] is not None:
            out_copies[c % 2].wait()
        out_copies[c % 2] = compute(c, bufs[c % 2], copies)
        copies = nxt
    for oc in out_copies:
        if oc is not None:
            oc.wait()


@functools.cache
def _sc_gather_kernel():
    mesh = plsc.VectorSubcoreMesh(
        core_axis_name="c", subcore_axis_name="s", num_cores=2, num_subcores=16)
    return pl.kernel(
        _sc_gather_impl,
        out_type=jax.ShapeDtypeStruct((NOUT * DH,), jnp.float32),
        mesh=mesh,
        scratch_types=[
            pltpu.VMEM((_NGATH, 128), jnp.int32),
            pltpu.VMEM((_NGATH, 128), jnp.int32),
            pltpu.VMEM((_CH * NPTS,), jnp.float32),
            pltpu.VMEM((_CH * NPTS,), jnp.float32),
            pltpu.VMEM((_CH * NPTS, DH), jnp.float32),
            pltpu.VMEM((_CH * NPTS, DH), jnp.float32),
            pltpu.VMEM((_CH * DH,), jnp.float32),
            pltpu.VMEM((_CH * DH,), jnp.float32),
            pltpu.SemaphoreType.DMA,
            pltpu.SemaphoreType.DMA,
            pltpu.SemaphoreType.DMA,
        ],
        compiler_params=pltpu.CompilerParams(use_tc_tiling_on_sc=False),
    )


_FBLK = 512


def _final_body(attn_ref, tgt_ref, w_ref, b_ref, g_ref, bn_ref, out_ref):
    y = (jnp.dot(attn_ref[:], w_ref[:], preferred_element_type=jnp.float32,
                 precision=lax.Precision.HIGHEST)
         + b_ref[:] + tgt_ref[:])
    m = jnp.mean(y, axis=1, keepdims=True)
    yc = y - m
    v = jnp.mean(yc * yc, axis=1, keepdims=True)
    out_ref[:] = yc / jnp.sqrt(v + 1e-5) * g_ref[:] + bn_ref[:]


def _final(attn_f, tgt_f, W_out, b_out, g1, b1n):
    grid = (B * LQ) // _FBLK
    return pl.pallas_call(
        _final_body,
        grid=(grid,),
        in_specs=[
            pl.BlockSpec((_FBLK, D), lambda i: (i, 0)),
            pl.BlockSpec((_FBLK, D), lambda i: (i, 0)),
            pl.BlockSpec((D, D), lambda i: (0, 0)),
            pl.BlockSpec((1, D), lambda i: (0, 0)),
            pl.BlockSpec((1, D), lambda i: (0, 0)),
            pl.BlockSpec((1, D), lambda i: (0, 0)),
        ],
        out_specs=pl.BlockSpec((_FBLK, D), lambda i: (i, 0)),
        out_shape=jax.ShapeDtypeStruct((B * LQ, D), jnp.float32),
    )(attn_f, tgt_f, W_out, b_out, g1, b1n)


def kernel(tgt, src, reference_points, spatial_shapes, level_start_index,
           W_off, b_off, W_attn, b_attn, W_val, b_val, W_out, b_out,
           g1, b1n, W1, bl1, W2, bl2, g2, b2n):
    tgt_f = tgt.reshape(B * LQ, D)
    src_f = src.reshape(B * LIN, D)
    rp_f = reference_points.reshape(B * LQ, 2)

    value = _valproj(src_f, W_val, b_val.reshape(1, D))
    w128, idx128 = _prep(tgt_f, rp_f, W_off, b_off.reshape(1, 64),
                         W_attn, b_attn.reshape(1, 32))

    table = value.reshape(B * LIN * H, DH)
    attn_flat = _sc_gather_kernel()(table, idx128, w128.reshape(-1))

    out = _final(attn_flat.reshape(B * LQ, D), tgt_f,
                 W_out, b_out.reshape(1, D), g1.reshape(1, D), b1n.reshape(1, D))
    return out.reshape(B, LQ, D)


# speedup vs baseline: 5.0858x; 1.1765x over previous
import functools

import jax
import jax.numpy as jnp
import numpy as np
from jax import lax
from jax.experimental import pallas as pl
from jax.experimental.pallas import tpu as pltpu
from jax.experimental.pallas import tpu_sc as plsc

B, LQ, D = 4, 1024, 256
H, P = 8, 4
HH, WW = 100, 100
LIN = HH * WW
DH = D // H
NPTS = P * 4
NOUT = B * LQ * H

_S = np.zeros((32, 32), np.float32)
for _i in range(32):
    for _j in range(32):
        if _i // 4 == _j // 4:
            _S[_i, _j] = 1.0
_A1 = np.zeros((32, 128), np.float32)
_MYX = np.zeros((128, 256), np.float32)
for _h in range(H):
    for _p in range(P):
        for _c in range(4):
            _l = _h * 16 + _p * 4 + _c
            _A1[_h * 4 + _p, _l] = 1.0
            _MYX[(_c >> 1) * 64 + _h * 8 + _p * 2 + 1, _l] = 1.0
            _MYX[(_c & 1) * 64 + _h * 8 + _p * 2 + 0, 128 + _l] = 1.0

_VBLK = 2000


def _valproj_body(src_ref, w_ref, b_ref, out_ref):
    r = (jnp.dot(src_ref[:], w_ref[:], preferred_element_type=jnp.float32,
                 precision=lax.Precision.HIGHEST)
         + b_ref[:])
    out_ref[:] = r.reshape(_VBLK * 2, 128)


def _valproj(src_f, W_val, b_val):
    grid = (B * LIN) // _VBLK
    return pl.pallas_call(
        _valproj_body,
        grid=(grid,),
        in_specs=[
            pl.BlockSpec((_VBLK, D), lambda i: (i, 0)),
            pl.BlockSpec((D, D), lambda i: (0, 0)),
            pl.BlockSpec((1, D), lambda i: (0, 0)),
        ],
        out_specs=pl.BlockSpec((_VBLK * 2, 128), lambda i: (i, 0)),
        out_shape=jax.ShapeDtypeStruct((B * LIN * 2, 128), jnp.float32),
    )(src_f, W_val, b_val)


_QBLK = 512


def _prep_body(tgt_ref, rp_ref, woff_ref, boff_ref, wattn_ref, battn_ref,
               s_ref, a1_ref, myx_ref, w_out_ref, idx_out_ref):
    i = pl.program_id(0)
    b_f = (i // (LQ // _QBLK)).astype(jnp.float32)
    t = tgt_ref[:]
    hi = lax.Precision.HIGHEST
    off = jnp.dot(t, woff_ref[:], preferred_element_type=jnp.float32, precision=hi) + boff_ref[:]
    a = jnp.dot(t, wattn_ref[:], preferred_element_type=jnp.float32, precision=hi) + battn_ref[:]
    a = a - jnp.max(a, axis=1, keepdims=True)
    ea = jnp.exp(a)
    gs = jnp.dot(ea, s_ref[:], preferred_element_type=jnp.float32, precision=hi)
    aw = ea / gs

    lane64 = lax.broadcasted_iota(jnp.int32, (_QBLK, 64), 1)
    is_x = (lane64 % 2) == 0
    rp = rp_ref[:]
    refc = jnp.where(is_x, rp[:, 0:1], rp[:, 1:2])
    loc = (refc + off / 100.0) * 100.0 - 0.5
    fl = jnp.floor(loc)
    w1 = loc - fl
    w0 = 1.0 - w1
    v0 = ((fl >= 0.0) & (fl <= 99.0)).astype(jnp.float32)
    v1 = ((fl >= -1.0) & (fl <= 98.0)).astype(jnp.float32)
    c0 = jnp.clip(fl, 0.0, 99.0)
    c1 = jnp.clip(fl + 1.0, 0.0, 99.0)

    MYX = myx_ref[:]
    wyx = jnp.dot(jnp.concatenate([w0, w1], axis=1), MYX,
                  preferred_element_type=jnp.float32, precision=hi)
    vyx = jnp.dot(jnp.concatenate([v0, v1], axis=1), MYX,
                  preferred_element_type=jnp.float32)
    cyx = jnp.dot(jnp.concatenate([c0, c1], axis=1), MYX,
                  preferred_element_type=jnp.float32)
    aw128 = jnp.dot(aw, a1_ref[:], preferred_element_type=jnp.float32, precision=hi)

    h128 = (lax.broadcasted_iota(jnp.int32, (_QBLK, 128), 1) // 16).astype(jnp.float32)
    base = b_f * float(LIN * H)
    idxf = (cyx[:, :128] * 100.0 + cyx[:, 128:]) * 8.0 + h128 + base
    w_out_ref[:] = (aw128 * wyx[:, :128] * wyx[:, 128:]
                    * vyx[:, :128] * vyx[:, 128:])
    idx_out_ref[:] = idxf.astype(jnp.int32)


def _prep(tgt_f, rp_f, W_off, b_off, W_attn, b_attn):
    grid = (B * LQ) // _QBLK
    return pl.pallas_call(
        _prep_body,
        grid=(grid,),
        in_specs=[
            pl.BlockSpec((_QBLK, D), lambda i: (i, 0)),
            pl.BlockSpec((_QBLK, 2), lambda i: (i, 0)),
            pl.BlockSpec((D, 64), lambda i: (0, 0)),
            pl.BlockSpec((1, 64), lambda i: (0, 0)),
            pl.BlockSpec((D, 32), lambda i: (0, 0)),
            pl.BlockSpec((1, 32), lambda i: (0, 0)),
            pl.BlockSpec((32, 32), lambda i: (0, 0)),
            pl.BlockSpec((32, 128), lambda i: (0, 0)),
            pl.BlockSpec((128, 256), lambda i: (0, 0)),
        ],
        out_specs=[
            pl.BlockSpec((_QBLK, 128), lambda i: (i, 0)),
            pl.BlockSpec((_QBLK, 128), lambda i: (i, 0)),
        ],
        out_shape=[
            jax.ShapeDtypeStruct((B * LQ, 128), jnp.float32),
            jax.ShapeDtypeStruct((B * LQ, 128), jnp.int32),
        ],
    )(tgt_f, rp_f, W_off, b_off, W_attn, b_attn, _S, _A1, _MYX)


_NW = 32
_RPW = NOUT // _NW
_CH = 64
_NCHUNK = _RPW // _CH
_NGATH = (_CH * NPTS) // 128

def _sc_gather_impl(table_hbm, idx_hbm, w_hbm, out_hbm,
                    idx_a, idx_b, w_a, w_b, rows_a, rows_b, out_a, out_b,
                    sem_a, sem_b, osem):
    wid = lax.axis_index("s") * 2 + lax.axis_index("c")
    table = table_hbm
    bufs = ((idx_a, w_a, rows_a, out_a, sem_a),
            (idx_b, w_b, rows_b, out_b, sem_b))

    def stage(c, buf):
        idx_v, w_v, rows_v, _, sem = buf
        base = wid * _RPW + c * _CH
        pltpu.sync_copy(idx_hbm.at[pl.ds(wid * 128 + c * _NGATH, _NGATH)], idx_v)
        pltpu.sync_copy(w_hbm.at[pl.ds(base * NPTS, _CH * NPTS)], w_v)
        return [pltpu.async_copy(table.at[idx_v.at[j]],
                                 rows_v.at[pl.ds(j * 128, 128)], sem)
                for j in range(_NGATH)]

    def compute(c, buf, copies):
        _, w_v, rows_v, out_v, _ = buf
        base = wid * _RPW + c * _CH
        for cp in copies:
            cp.wait()

        def j_body(j, _):
            wv = w_v[pl.ds(j * NPTS, 16)]
            acc0 = jnp.zeros((16,), jnp.float32)
            acc1 = jnp.zeros((16,), jnp.float32)
            for i in range(NPTS):
                k = j * NPTS + i
                g0 = rows_v[k, pl.ds(0, 16)]
                g1 = rows_v[k, pl.ds(16, 16)]
                s = wv[i]
                acc0 = acc0 + g0 * s
                acc1 = acc1 + g1 * s
            out_v[pl.ds(j * DH, 16)] = acc0
            out_v[pl.ds(j * DH + 16, 16)] = acc1
            return 0

        lax.fori_loop(0, _CH, j_body, 0)
        return pltpu.async_copy(out_v, out_hbm.at[pl.ds(base * DH, _CH * DH)], osem)

    out_copies = [None, None]
    copies = stage(0, bufs[0])
    for c in range(_NCHUNK):
        nxt = stage(c + 1, bufs[(c + 1) % 2]) if c + 1 < _NCHUNK else None
        if out_copies[c % 2] is not None:
            out_copies[c % 2].wait()
        out_copies[c % 2] = compute(c, bufs[c % 2], copies)
        copies = nxt
    for oc in out_copies:
        if oc is not None:
            oc.wait()


@functools.cache
def _sc_gather_kernel():
    mesh = plsc.VectorSubcoreMesh(
        core_axis_name="c", subcore_axis_name="s", num_cores=2, num_subcores=16)
    return pl.kernel(
        _sc_gather_impl,
        out_type=jax.ShapeDtypeStruct((NOUT * DH,), jnp.float32),
        mesh=mesh,
        scratch_types=[
            pltpu.VMEM((_NGATH, 128), jnp.int32),
            pltpu.VMEM((_NGATH, 128), jnp.int32),
            pltpu.VMEM((_CH * NPTS,), jnp.float32),
            pltpu.VMEM((_CH * NPTS,), jnp.float32),
            pltpu.VMEM((_CH * NPTS, DH), jnp.float32),
            pltpu.VMEM((_CH * NPTS, DH), jnp.float32),
            pltpu.VMEM((_CH * DH,), jnp.float32),
            pltpu.VMEM((_CH * DH,), jnp.float32),
            pltpu.SemaphoreType.DMA,
            pltpu.SemaphoreType.DMA,
            pltpu.SemaphoreType.DMA,
        ],
        compiler_params=pltpu.CompilerParams(use_tc_tiling_on_sc=False),
    )


_FBLK = 512


def _final_body(attn_ref, tgt_ref, w_ref, b_ref, g_ref, bn_ref, out_ref):
    y = (jnp.dot(attn_ref[:], w_ref[:], preferred_element_type=jnp.float32,
                 precision=lax.Precision.HIGHEST)
         + b_ref[:] + tgt_ref[:])
    m = jnp.mean(y, axis=1, keepdims=True)
    yc = y - m
    v = jnp.mean(yc * yc, axis=1, keepdims=True)
    out_ref[:] = yc / jnp.sqrt(v + 1e-5) * g_ref[:] + bn_ref[:]


def _final(attn_f, tgt_f, W_out, b_out, g1, b1n):
    grid = (B * LQ) // _FBLK
    return pl.pallas_call(
        _final_body,
        grid=(grid,),
        in_specs=[
            pl.BlockSpec((_FBLK, D), lambda i: (i, 0)),
            pl.BlockSpec((_FBLK, D), lambda i: (i, 0)),
            pl.BlockSpec((D, D), lambda i: (0, 0)),
            pl.BlockSpec((1, D), lambda i: (0, 0)),
            pl.BlockSpec((1, D), lambda i: (0, 0)),
            pl.BlockSpec((1, D), lambda i: (0, 0)),
        ],
        out_specs=pl.BlockSpec((_FBLK, D), lambda i: (i, 0)),
        out_shape=jax.ShapeDtypeStruct((B * LQ, D), jnp.float32),
    )(attn_f, tgt_f, W_out, b_out, g1, b1n)


def kernel(tgt, src, reference_points, spatial_shapes, level_start_index,
           W_off, b_off, W_attn, b_attn, W_val, b_val, W_out, b_out,
           g1, b1n, W1, bl1, W2, bl2, g2, b2n):
    tgt_f = tgt.reshape(B * LQ, D)
    src_f = src.reshape(B * LIN, D)
    rp_f = reference_points.reshape(B * LQ, 2)

    value = _valproj(src_f, W_val, b_val.reshape(1, D))
    w128, idx128 = _prep(tgt_f, rp_f, W_off, b_off.reshape(1, 64),
                         W_attn, b_attn.reshape(1, 32))

    attn_flat = _sc_gather_kernel()(value.reshape(B * LIN * H, DH), idx128, w128.reshape(-1))

    out = _final(attn_flat.reshape(B * LQ, D), tgt_f,
                 W_out, b_out.reshape(1, D), g1.reshape(1, D), b1n.reshape(1, D))
    return out.reshape(B, LQ, D)


# speedup vs baseline: 5.1527x; 1.0131x over previous
import functools

import jax
import jax.numpy as jnp
import numpy as np
from jax import lax
from jax.experimental import pallas as pl
from jax.experimental.pallas import tpu as pltpu
from jax.experimental.pallas import tpu_sc as plsc

B, LQ, D = 4, 1024, 256
H, P = 8, 4
HH, WW = 100, 100
LIN = HH * WW
DH = D // H
NPTS = P * 4
NOUT = B * LQ * H

_S = np.zeros((32, 32), np.float32)
for _i in range(32):
    for _j in range(32):
        if _i // 4 == _j // 4:
            _S[_i, _j] = 1.0
_A1 = np.zeros((32, 128), np.float32)
_MYX = np.zeros((128, 256), np.float32)
for _h in range(H):
    for _p in range(P):
        for _c in range(4):
            _l = _h * 16 + _p * 4 + _c
            _A1[_h * 4 + _p, _l] = 1.0
            _MYX[(_c >> 1) * 64 + _h * 8 + _p * 2 + 1, _l] = 1.0
            _MYX[(_c & 1) * 64 + _h * 8 + _p * 2 + 0, 128 + _l] = 1.0

_VBLK = 2000


def _valproj_body(src_ref, w_ref, b_ref, out_ref):
    r = (jnp.dot(src_ref[:], w_ref[:], preferred_element_type=jnp.float32,
                 precision=lax.Precision.HIGHEST)
         + b_ref[:])
    out_ref[:] = r.reshape(_VBLK * 2, 128)


def _valproj(src_f, W_val, b_val):
    grid = (B * LIN) // _VBLK
    return pl.pallas_call(
        _valproj_body,
        grid=(grid,),
        in_specs=[
            pl.BlockSpec((_VBLK, D), lambda i: (i, 0)),
            pl.BlockSpec((D, D), lambda i: (0, 0)),
            pl.BlockSpec((1, D), lambda i: (0, 0)),
        ],
        out_specs=pl.BlockSpec((_VBLK * 2, 128), lambda i: (i, 0)),
        out_shape=jax.ShapeDtypeStruct((B * LIN * 2, 128), jnp.float32),
    )(src_f, W_val, b_val)


_QBLK = 512


def _prep_body(tgt_ref, rp_ref, woff_ref, boff_ref, wattn_ref, battn_ref,
               s_ref, a1_ref, myx_ref, w_out_ref, idx_out_ref):
    i = pl.program_id(0)
    b_f = (i // (LQ // _QBLK)).astype(jnp.float32)
    t = tgt_ref[:]
    hi = lax.Precision.HIGHEST
    off = jnp.dot(t, woff_ref[:], preferred_element_type=jnp.float32, precision=hi) + boff_ref[:]
    a = jnp.dot(t, wattn_ref[:], preferred_element_type=jnp.float32, precision=hi) + battn_ref[:]
    a = a - jnp.max(a, axis=1, keepdims=True)
    ea = jnp.exp(a)
    gs = jnp.dot(ea, s_ref[:], preferred_element_type=jnp.float32, precision=hi)
    aw = ea / gs

    lane64 = lax.broadcasted_iota(jnp.int32, (_QBLK, 64), 1)
    is_x = (lane64 % 2) == 0
    rp = rp_ref[:]
    refc = jnp.where(is_x, rp[:, 0:1], rp[:, 1:2])
    loc = (refc + off / 100.0) * 100.0 - 0.5
    fl = jnp.floor(loc)
    w1 = loc - fl
    w0 = 1.0 - w1
    v0 = ((fl >= 0.0) & (fl <= 99.0)).astype(jnp.float32)
    v1 = ((fl >= -1.0) & (fl <= 98.0)).astype(jnp.float32)
    c0 = jnp.clip(fl, 0.0, 99.0)
    c1 = jnp.clip(fl + 1.0, 0.0, 99.0)

    MYX = myx_ref[:]
    wyx = jnp.dot(jnp.concatenate([w0, w1], axis=1), MYX,
                  preferred_element_type=jnp.float32, precision=hi)
    vyx = jnp.dot(jnp.concatenate([v0, v1], axis=1), MYX,
                  preferred_element_type=jnp.float32)
    cyx = jnp.dot(jnp.concatenate([c0, c1], axis=1), MYX,
                  preferred_element_type=jnp.float32)
    aw128 = jnp.dot(aw, a1_ref[:], preferred_element_type=jnp.float32, precision=hi)

    h128 = (lax.broadcasted_iota(jnp.int32, (_QBLK, 128), 1) // 16).astype(jnp.float32)
    base = b_f * float(LIN * H)
    idxf = (cyx[:, :128] * 100.0 + cyx[:, 128:]) * 8.0 + h128 + base
    w_out_ref[:] = (aw128 * wyx[:, :128] * wyx[:, 128:]
                    * vyx[:, :128] * vyx[:, 128:])
    idx_out_ref[:] = idxf.astype(jnp.int32)


def _prep(tgt_f, rp_f, W_off, b_off, W_attn, b_attn):
    grid = (B * LQ) // _QBLK
    return pl.pallas_call(
        _prep_body,
        grid=(grid,),
        in_specs=[
            pl.BlockSpec((_QBLK, D), lambda i: (i, 0)),
            pl.BlockSpec((_QBLK, 2), lambda i: (i, 0)),
            pl.BlockSpec((D, 64), lambda i: (0, 0)),
            pl.BlockSpec((1, 64), lambda i: (0, 0)),
            pl.BlockSpec((D, 32), lambda i: (0, 0)),
            pl.BlockSpec((1, 32), lambda i: (0, 0)),
            pl.BlockSpec((32, 32), lambda i: (0, 0)),
            pl.BlockSpec((32, 128), lambda i: (0, 0)),
            pl.BlockSpec((128, 256), lambda i: (0, 0)),
        ],
        out_specs=[
            pl.BlockSpec((_QBLK, 128), lambda i: (i, 0)),
            pl.BlockSpec((_QBLK, 128), lambda i: (i, 0)),
        ],
        out_shape=[
            jax.ShapeDtypeStruct((B * LQ, 128), jnp.float32),
            jax.ShapeDtypeStruct((B * LQ, 128), jnp.int32),
        ],
    )(tgt_f, rp_f, W_off, b_off, W_attn, b_attn, _S, _A1, _MYX)


_NW = 32
_RPW = NOUT // _NW
_CH = 64
_NCHUNK = _RPW // _CH
_NGATH = (_CH * NPTS) // 128

def _sc_gather_impl(table_hbm, idx_hbm, w_hbm, out_hbm,
                    idx_a, idx_b, w_a, w_b, rows_a, rows_b, out_a, out_b,
                    sem_a, sem_b, osem):
    wid = lax.axis_index("s") * 2 + lax.axis_index("c")
    table = table_hbm
    bufs = ((idx_a, w_a, rows_a, out_a, sem_a),
            (idx_b, w_b, rows_b, out_b, sem_b))

    def stage(c, buf):
        idx_v, w_v, rows_v, _, sem = buf
        base = wid * _RPW + c * _CH
        pltpu.sync_copy(idx_hbm.at[pl.ds(wid * 128 + c * _NGATH, _NGATH)], idx_v)
        pltpu.sync_copy(w_hbm.at[pl.ds(base * NPTS, _CH * NPTS)], w_v)
        return [pltpu.async_copy(table.at[idx_v.at[j]],
                                 rows_v.at[pl.ds(j * 128, 128)], sem)
                for j in range(_NGATH)]

    def compute(c, buf, copies):
        _, w_v, rows_v, out_v, _ = buf
        base = wid * _RPW + c * _CH
        for cp in copies:
            cp.wait()

        @plsc.parallel_loop(0, _CH, 1, unroll=2)
        def j_body(j):
            wv = w_v[pl.ds(j * NPTS, 16)]
            acc0 = jnp.zeros((16,), jnp.float32)
            acc1 = jnp.zeros((16,), jnp.float32)
            for i in range(NPTS):
                k = j * NPTS + i
                g0 = rows_v[k, pl.ds(0, 16)]
                g1 = rows_v[k, pl.ds(16, 16)]
                s = wv[i]
                acc0 = acc0 + g0 * s
                acc1 = acc1 + g1 * s
            out_v[pl.ds(j * DH, 16)] = acc0
            out_v[pl.ds(j * DH + 16, 16)] = acc1
        return pltpu.async_copy(out_v, out_hbm.at[pl.ds(base * DH, _CH * DH)], osem)

    out_copies = [None, None]
    copies = stage(0, bufs[0])
    for c in range(_NCHUNK):
        nxt = stage(c + 1, bufs[(c + 1) % 2]) if c + 1 < _NCHUNK else None
        if out_copies[c % 2] is not None:
            out_copies[c % 2].wait()
        out_copies[c % 2] = compute(c, bufs[c % 2], copies)
        copies = nxt
    for oc in out_copies:
        if oc is not None:
            oc.wait()


@functools.cache
def _sc_gather_kernel():
    mesh = plsc.VectorSubcoreMesh(
        core_axis_name="c", subcore_axis_name="s", num_cores=2, num_subcores=16)
    return pl.kernel(
        _sc_gather_impl,
        out_type=jax.ShapeDtypeStruct((NOUT * DH,), jnp.float32),
        mesh=mesh,
        scratch_types=[
            pltpu.VMEM((_NGATH, 128), jnp.int32),
            pltpu.VMEM((_NGATH, 128), jnp.int32),
            pltpu.VMEM((_CH * NPTS,), jnp.float32),
            pltpu.VMEM((_CH * NPTS,), jnp.float32),
            pltpu.VMEM((_CH * NPTS, DH), jnp.float32),
            pltpu.VMEM((_CH * NPTS, DH), jnp.float32),
            pltpu.VMEM((_CH * DH,), jnp.float32),
            pltpu.VMEM((_CH * DH,), jnp.float32),
            pltpu.SemaphoreType.DMA,
            pltpu.SemaphoreType.DMA,
            pltpu.SemaphoreType.DMA,
        ],
        compiler_params=pltpu.CompilerParams(use_tc_tiling_on_sc=False),
    )


_FBLK = 512


def _final_body(attn_ref, tgt_ref, w_ref, b_ref, g_ref, bn_ref, out_ref):
    y = (jnp.dot(attn_ref[:], w_ref[:], preferred_element_type=jnp.float32,
                 precision=lax.Precision.HIGHEST)
         + b_ref[:] + tgt_ref[:])
    m = jnp.mean(y, axis=1, keepdims=True)
    yc = y - m
    v = jnp.mean(yc * yc, axis=1, keepdims=True)
    out_ref[:] = yc / jnp.sqrt(v + 1e-5) * g_ref[:] + bn_ref[:]


def _final(attn_f, tgt_f, W_out, b_out, g1, b1n):
    grid = (B * LQ) // _FBLK
    return pl.pallas_call(
        _final_body,
        grid=(grid,),
        in_specs=[
            pl.BlockSpec((_FBLK, D), lambda i: (i, 0)),
            pl.BlockSpec((_FBLK, D), lambda i: (i, 0)),
            pl.BlockSpec((D, D), lambda i: (0, 0)),
            pl.BlockSpec((1, D), lambda i: (0, 0)),
            pl.BlockSpec((1, D), lambda i: (0, 0)),
            pl.BlockSpec((1, D), lambda i: (0, 0)),
        ],
        out_specs=pl.BlockSpec((_FBLK, D), lambda i: (i, 0)),
        out_shape=jax.ShapeDtypeStruct((B * LQ, D), jnp.float32),
    )(attn_f, tgt_f, W_out, b_out, g1, b1n)


def kernel(tgt, src, reference_points, spatial_shapes, level_start_index,
           W_off, b_off, W_attn, b_attn, W_val, b_val, W_out, b_out,
           g1, b1n, W1, bl1, W2, bl2, g2, b2n):
    tgt_f = tgt.reshape(B * LQ, D)
    src_f = src.reshape(B * LIN, D)
    rp_f = reference_points.reshape(B * LQ, 2)

    value = _valproj(src_f, W_val, b_val.reshape(1, D))
    w128, idx128 = _prep(tgt_f, rp_f, W_off, b_off.reshape(1, 64),
                         W_attn, b_attn.reshape(1, 32))

    attn_flat = _sc_gather_kernel()(value.reshape(B * LIN * H, DH), idx128, w128.reshape(-1))

    out = _final(attn_flat.reshape(B * LQ, D), tgt_f,
                 W_out, b_out.reshape(1, D), g1.reshape(1, D), b1n.reshape(1, D))
    return out.reshape(B, LQ, D)
